# baseline (speedup 1.0000x reference)
import functools

import jax
import jax.numpy as jnp
from jax import lax
from jax.experimental import pallas as pl
from jax.experimental.pallas import tpu as pltpu
from jax.experimental.pallas import tpu_sc as plsc

D = 64
K = 20
UROWS = K + 1
NW = 32
CH = 32
DT = D // 16
G = 128

_GDN = lax.GatherDimensionNumbers(
    offset_dims=(), collapsed_slice_dims=(0,), start_index_map=(0,))


def _reg_gather(v, idx):
    return lax.gather(v, idx[:, None], _GDN, (1,),
                      mode=lax.GatherScatterMode.PROMISE_IN_BOUNDS)


def _hsum(v, perms):
    for p in perms:
        v = v + _reg_gather(v, p)
    return v


def _sc_scores(cidx4, uidx4, v4, u4):
    B = cidx4.shape[0] // DT
    per_w = B // NW
    nch = per_w // CH
    cn = CH * DT
    un = CH * UROWS * DT
    mesh = plsc.VectorSubcoreMesh(core_axis_name="c", subcore_axis_name="s",
                                  num_cores=2, num_subcores=16)

    @functools.partial(
        pl.kernel,
        out_type=[jax.ShapeDtypeStruct((B,), jnp.float32),
                  jax.ShapeDtypeStruct((B,), jnp.float32)],
        mesh=mesh,
        scratch_types=[
            pltpu.VMEM((cn,), jnp.int32),
            pltpu.VMEM((un,), jnp.int32),
            pltpu.VMEM((cn, 16), jnp.float32),
            pltpu.VMEM((un, 16), jnp.float32),
            pltpu.VMEM((CH,), jnp.float32),
            pltpu.VMEM((CH,), jnp.float32),
            pltpu.SemaphoreType.DMA,
        ],
        compiler_params=pltpu.CompilerParams(use_tc_tiling_on_sc=False),
    )
    def k(v_hbm, u_hbm, cidx_hbm, uidx_hbm, pos_hbm, neg_hbm,
          cidx_v, uidx_v, crow_v, urow_v, pos_v, neg_v, sem):
        wid = lax.axis_index("s") * 2 + lax.axis_index("c")
        base_w = wid * per_w
        lanes = lax.iota(jnp.int32, 16)
        perms = [lanes ^ s for s in (1, 2, 4, 8)]

        def chunk_body(ch, carry):
            base = base_w + ch * CH
            pltpu.sync_copy(cidx_hbm.at[pl.ds(base * DT, cn)], cidx_v)
            pltpu.sync_copy(uidx_hbm.at[pl.ds(base * UROWS * DT, un)],
                            uidx_v)
            copies = [pltpu.async_copy(v_hbm.at[cidx_v], crow_v, sem)]
            for g in range(un // G):
                copies.append(pltpu.async_copy(
                    u_hbm.at[uidx_v.at[pl.ds(g * G, G)]],
                    urow_v.at[pl.ds(g * G, G)], sem))
            for cpy in copies:
                cpy.wait()

            zero16 = jnp.zeros((16,), jnp.float32)

            def group_body(gr, c3):
                jbase = gr * 16

                def item_body(l, acc):
                    accp, accn = acc
                    j = jbase + l
                    c = [crow_v[j * DT + t, :] for t in range(DT)]
                    fb = j * UROWS * DT
                    tg = [urow_v[fb + t, :] for t in range(DT)]
                    ap = c[0] * tg[0]
                    for t in range(1, DT):
                        ap = ap + c[t] * tg[t]
                    ns = [urow_v[fb + DT + t, :] for t in range(DT)]
                    for kk in range(2, UROWS):
                        for t in range(DT):
                            ns[t] = ns[t] + urow_v[fb + kk * DT + t, :]
                    an = c[0] * ns[0]
                    for t in range(1, DT):
                        an = an + c[t] * ns[t]
                    accp = jnp.where(lanes == l, _hsum(ap, perms), accp)
                    accn = jnp.where(lanes == l, _hsum(an, perms), accn)
                    return accp, accn

                accp, accn = lax.fori_loop(0, 16, item_body,
                                           (zero16, zero16))
                pos_v[pl.ds(jbase, 16)] = accp
                neg_v[pl.ds(jbase, 16)] = -accn
                return c3

            lax.fori_loop(0, CH // 16, group_body, 0)
            pltpu.sync_copy(pos_v, pos_hbm.at[pl.ds(base, CH)])
            pltpu.sync_copy(neg_v, neg_hbm.at[pl.ds(base, CH)])
            return carry

        lax.fori_loop(0, nch, chunk_body, 0)

    return k(v4, u4, cidx4, uidx4)


def _tc_loss(pos2d, neg2d):
    n = pos2d.shape[0] * pos2d.shape[1]

    def body(p_ref, n_ref, o_ref):
        def logsig(x):
            return jnp.minimum(x, 0.0) - jnp.log1p(jnp.exp(-jnp.abs(x)))

        tot = jnp.sum(logsig(p_ref[...]) + logsig(n_ref[...]))
        o_ref[0, 0] = -tot / n

    return pl.pallas_call(
        body,
        out_shape=jax.ShapeDtypeStruct((1, 1), jnp.float32),
        out_specs=pl.BlockSpec(memory_space=pltpu.SMEM),
    )(pos2d, neg2d)


def _expand4(idx):
    return (idx[:, None] * DT + jnp.arange(DT, dtype=jnp.int32)).reshape(-1)


@jax.jit
def kernel(center_words, target_words, negative_words, embedding_v, embedding_u):
    B = center_words.shape[0]
    cidx4 = _expand4(center_words.reshape(B).astype(jnp.int32))
    uidx = jnp.concatenate(
        [target_words.astype(jnp.int32), negative_words.astype(jnp.int32)],
        axis=1).reshape(-1)
    uidx4 = _expand4(uidx)
    v4 = embedding_v.reshape(-1, 16)
    u4 = embedding_u.reshape(-1, 16)
    pos, neg = _sc_scores(cidx4, uidx4, v4, u4)
    loss = _tc_loss(pos.reshape(128, -1), neg.reshape(128, -1))
    return loss[0, 0]
